# baseline (speedup 1.0000x reference)
import functools

import jax
import jax.numpy as jnp
from jax import lax
from jax.experimental import pallas as pl
from jax.experimental.pallas import tpu as pltpu
from jax.experimental.pallas import tpu_sc as plsc

N = 10000
N_PAD = 10240
E = 320000
D_IN = 128
D_OUT = 128

NC = 2
NS = 16
LANES = 16

CHUNK = 128
E_PAD = 327680
EDGES_PER_TILE = E_PAD // (NC * NS)
NCHUNKS = EDGES_PER_TILE // CHUNK
ROWS_PER_TILE = N_PAD // NS
ZBLK = 64

MM_BLK = 1024


def _matmul_body(x_ref, w_ref, o_ref):
    o_ref[...] = jnp.dot(x_ref[...], w_ref[...], preferred_element_type=jnp.float32)


def _tc_matmul(x, W):
    return pl.pallas_call(
        _matmul_body,
        grid=(N_PAD // MM_BLK,),
        in_specs=[
            pl.BlockSpec((MM_BLK, D_IN), lambda i: (i, 0)),
            pl.BlockSpec((D_IN, D_OUT), lambda i: (0, 0)),
        ],
        out_specs=pl.BlockSpec((MM_BLK, D_OUT), lambda i: (i, 0)),
        out_shape=jax.ShapeDtypeStruct((N_PAD, D_OUT), jnp.float32),
    )(x, W)


def _add_body(a_ref, b_ref, o_ref):
    o_ref[...] = a_ref[...] + b_ref[...]


def _tc_add(a, b):
    return pl.pallas_call(
        _add_body,
        grid=(N_PAD // MM_BLK,),
        in_specs=[
            pl.BlockSpec((MM_BLK, D_OUT), lambda i: (i, 0)),
            pl.BlockSpec((MM_BLK, D_OUT), lambda i: (i, 0)),
        ],
        out_specs=pl.BlockSpec((MM_BLK, D_OUT), lambda i: (i, 0)),
        out_shape=jax.ShapeDtypeStruct((N_PAD, D_OUT), jnp.float32),
    )(a, b)


def _bcast_lane(v, i):
    idx = jnp.full((LANES,), i, dtype=jnp.int32)
    return lax.gather(
        v,
        idx[:, None],
        dimension_numbers=lax.GatherDimensionNumbers(
            offset_dims=(), collapsed_slice_dims=(0,), start_index_map=(0,)
        ),
        slice_sizes=(1,),
        mode=lax.GatherScatterMode.PROMISE_IN_BOUNDS,
    )


def _sc_spmm_body(
    ps, pk_hbm, vals_hbm, out0, out1,
    pkA, pkB, valsA, valsB, bufA, bufB, zbuf, acc,
    gsemA, gsemB, ssemA, ssemB, isemA, isemB, wsem,
):
    c = lax.axis_index("c")
    s = lax.axis_index("s")
    ebase = (c * NS + s) * EDGES_PER_TILE

    def idx_load(i, pk, vals, isem):
        base = ebase + i * CHUNK
        pltpu.async_copy(pk_hbm.at[:, pl.ds(base, CHUNK)], pk, isem)
        pltpu.async_copy(vals_hbm.at[pl.ds(base, CHUNK)], vals, isem)

    def idx_wait(pk, vals, isem):
        pltpu.make_async_copy(pk_hbm.at[:, pl.ds(0, CHUNK)], pk, isem).wait()
        pltpu.make_async_copy(vals_hbm.at[pl.ds(0, CHUNK)], vals, isem).wait()

    def gather(i, pk, buf, gsem):
        del i
        pltpu.async_copy(ps.at[pk.at[1]], buf, gsem)

    def gather_wait(buf, gsem):
        pltpu.make_async_copy(ps.at[pl.ds(0, CHUNK)], buf, gsem).wait()

    def scatter(pk, buf, ssem):
        pltpu.async_copy(buf, acc.at[pk.at[0]], ssem, add=True)

    def scatter_wait(buf, ssem):
        pltpu.make_async_copy(buf, acc.at[pl.ds(0, CHUNK)], ssem).wait()

    def scale(buf, vals):
        for g in range(CHUNK // LANES):
            vv = vals[pl.ds(g * LANES, LANES)]
            for i in range(LANES):
                e = g * LANES + i
                vb = _bcast_lane(vv, i)
                for j in range(D_OUT // LANES):
                    sl = pl.ds(j * LANES, LANES)
                    buf[e, sl] = buf[e, sl] * vb

    zero16 = jnp.zeros((LANES,), jnp.float32)

    def zrow(i, carry):
        for j in range(D_OUT // LANES):
            zbuf[i, pl.ds(j * LANES, LANES)] = zero16
        return carry

    lax.fori_loop(0, ZBLK, zrow, 0)
    row0 = s * ROWS_PER_TILE
    for b in range(ROWS_PER_TILE // ZBLK):
        pltpu.async_copy(zbuf, acc.at[pl.ds(row0 + b * ZBLK, ZBLK)], wsem)
    for b in range(ROWS_PER_TILE // ZBLK):
        pltpu.make_async_copy(zbuf, acc.at[pl.ds(row0, ZBLK)], wsem).wait()
    plsc.subcore_barrier()

    idx_load(0, pkA, valsA, isemA)
    idx_wait(pkA, valsA, isemA)
    gather(0, pkA, bufA, gsemA)
    idx_load(1, pkB, valsB, isemB)
    gather_wait(bufA, gsemA)
    scale(bufA, valsA)
    scatter(pkA, bufA, ssemA)
    idx_wait(pkB, valsB, isemB)
    gather(1, pkB, bufB, gsemB)

    def half(i, pkX, valsX, bufX, gsemX, ssemX, pkY, valsY, bufY, gsemY, ssemY, isemY):
        scatter_wait(bufY, ssemY)
        idx_load(i + 1, pkY, valsY, isemY)
        gather_wait(bufX, gsemX)
        scale(bufX, valsX)
        scatter(pkX, bufX, ssemX)
        idx_wait(pkY, valsY, isemY)
        gather(i + 1, pkY, bufY, gsemY)

    def body(k, carry):
        i0 = 2 * k + 1
        half(i0, pkB, valsB, bufB, gsemB, ssemB, pkA, valsA, bufA, gsemA, ssemA, isemA)
        half(i0 + 1, pkA, valsA, bufA, gsemA, ssemA, pkB, valsB, bufB, gsemB, ssemB, isemB)
        return carry

    lax.fori_loop(0, NCHUNKS // 2 - 1, body, 0)

    gather_wait(bufB, gsemB)
    scale(bufB, valsB)
    scatter(pkB, bufB, ssemB)
    scatter_wait(bufA, ssemA)
    scatter_wait(bufB, ssemB)
    plsc.subcore_barrier()

    @pl.when(c == 0)
    def _():
        for b in range(ROWS_PER_TILE // ZBLK):
            r = row0 + b * ZBLK
            pltpu.async_copy(acc.at[pl.ds(r, ZBLK)], out0.at[pl.ds(r, ZBLK)], wsem)
        for b in range(ROWS_PER_TILE // ZBLK):
            pltpu.make_async_copy(acc.at[pl.ds(row0, ZBLK)], out0.at[pl.ds(row0, ZBLK)], wsem).wait()

    @pl.when(c == 1)
    def _():
        for b in range(ROWS_PER_TILE // ZBLK):
            r = row0 + b * ZBLK
            pltpu.async_copy(acc.at[pl.ds(r, ZBLK)], out1.at[pl.ds(r, ZBLK)], wsem)
        for b in range(ROWS_PER_TILE // ZBLK):
            pltpu.make_async_copy(acc.at[pl.ds(row0, ZBLK)], out1.at[pl.ds(row0, ZBLK)], wsem).wait()


_sc_spmm = functools.partial(
    pl.kernel,
    mesh=plsc.VectorSubcoreMesh(core_axis_name="c", subcore_axis_name="s"),
    out_type=[
        jax.ShapeDtypeStruct((N_PAD, D_OUT), jnp.float32),
        jax.ShapeDtypeStruct((N_PAD, D_OUT), jnp.float32),
    ],
    scratch_types=[
        pltpu.VMEM((2, CHUNK), jnp.int32),
        pltpu.VMEM((2, CHUNK), jnp.int32),
        pltpu.VMEM((CHUNK,), jnp.float32),
        pltpu.VMEM((CHUNK,), jnp.float32),
        pltpu.VMEM((CHUNK, D_OUT), jnp.float32),
        pltpu.VMEM((CHUNK, D_OUT), jnp.float32),
        pltpu.VMEM((ZBLK, D_OUT), jnp.float32),
        pltpu.VMEM_SHARED((N_PAD, D_OUT), jnp.float32),
        pltpu.SemaphoreType.DMA,
        pltpu.SemaphoreType.DMA,
        pltpu.SemaphoreType.DMA,
        pltpu.SemaphoreType.DMA,
        pltpu.SemaphoreType.DMA,
        pltpu.SemaphoreType.DMA,
        pltpu.SemaphoreType.DMA,
    ],
)(_sc_spmm_body)


def kernel(x, adj_indices, adj_values, W):
    x_pad = jnp.pad(x, ((0, N_PAD - N), (0, 0)))
    ps = _tc_matmul(x_pad, W)
    pk = jnp.pad(adj_indices, ((0, 0), (0, E_PAD - E)))
    vals = jnp.pad(adj_values, (0, E_PAD - E))
    p0, p1 = _sc_spmm(ps, pk, vals)
    return _tc_add(p0, p1)[:N]
